# baseline (speedup 1.0000x reference)
import jax
import jax.numpy as jnp
from jax.experimental import pallas as pl
from jax.experimental.pallas import tpu as pltpu

N = 2048
D_MODEL = 256
OUT_DIM = 32
NDW = 48
PR = 64
ALPHA = 0.2
BR = 512


def _proj_kernel(x_ref, w_ref, a_ref, params_ref):
    whT = jax.lax.dot_general(
        w_ref[...], x_ref[...], (((0,), (1,)), ((), ())),
        preferred_element_type=jnp.float32,
    )
    st = jax.lax.dot_general(
        a_ref[...], whT, (((0,), (0,)), ((), ())),
        preferred_element_type=jnp.float32,
    )
    sp = st[0:1, :]
    tp = st[1:2, :]
    row = jax.lax.broadcasted_iota(jnp.int32, (PR, N), 0)
    params = jnp.where(
        row < OUT_DIM,
        jnp.pad(whT, ((0, PR - OUT_DIM), (0, 0))),
        jnp.where(row == OUT_DIM, 1.0, 0.0),
    )
    vqk = jnp.concatenate(
        [jnp.exp2(tp), jnp.exp2(ALPHA * tp), jnp.exp2((ALPHA - 1.0) * sp)],
        axis=0,
    )
    params = jnp.where(
        (row >= NDW) & (row < NDW + 3),
        jnp.pad(vqk, ((NDW, PR - NDW - 3), (0, 0))),
        params,
    )
    params_ref[...] = params.astype(jnp.bfloat16)


def _gat_kernel(adj_ref, params_ref, out_ref):
    i = pl.program_id(0)
    v = params_ref[NDW : NDW + 1, :]
    q = params_ref[NDW + 1 : NDW + 2, :]
    k = jnp.reshape(
        params_ref[NDW + 2 : NDW + 3, pl.ds(i * BR, BR)], (BR, 1)
    )
    e = jnp.minimum(v, k * q) * adj_ref[...].astype(jnp.bfloat16)
    nd = jax.lax.dot_general(
        e, params_ref[:NDW, :], (((1,), (1,)), ((), ())),
        preferred_element_type=jnp.float32,
    )
    r = nd[:, :OUT_DIM] / nd[:, OUT_DIM : OUT_DIM + 1]
    out_ref[...] = jnp.where(r > 0.0, r, jnp.exp(jnp.minimum(r, 0.0)) - 1.0)


def kernel(input, adj_mat, weights, a_values):
    a_cols = a_values.reshape(2, OUT_DIM).T * (-1.4426950408889634)

    params = pl.pallas_call(
        _proj_kernel,
        out_shape=jax.ShapeDtypeStruct((PR, N), jnp.bfloat16),
    )(input, weights, a_cols)

    out = pl.pallas_call(
        _gat_kernel,
        grid=(N // BR,),
        in_specs=[
            pl.BlockSpec((BR, N), lambda i: (i, 0)),
            pl.BlockSpec((PR, N), lambda i: (0, 0)),
        ],
        out_specs=pl.BlockSpec((BR, OUT_DIM), lambda i: (i, 0)),
        out_shape=jax.ShapeDtypeStruct((N, OUT_DIM), jnp.float32),
        compiler_params=pltpu.CompilerParams(
            dimension_semantics=("arbitrary",)
        ),
    )(adj_mat, params)
    return out


# speedup vs baseline: 1.0189x; 1.0189x over previous
import jax
import jax.numpy as jnp
from jax.experimental import pallas as pl
from jax.experimental.pallas import tpu as pltpu

N = 2048
D_MODEL = 256
OUT_DIM = 32
NDW = 48
PR = 64
ALPHA = 0.2
BR = 1024


def _proj_kernel(x_ref, w_ref, a_ref, params_ref):
    whT = jax.lax.dot_general(
        w_ref[...], x_ref[...], (((0,), (1,)), ((), ())),
        preferred_element_type=jnp.float32,
    )
    st = jax.lax.dot_general(
        a_ref[...], whT, (((0,), (0,)), ((), ())),
        preferred_element_type=jnp.float32,
    )
    sp = st[0:1, :]
    tp = st[1:2, :]
    row = jax.lax.broadcasted_iota(jnp.int32, (PR, N), 0)
    params = jnp.where(
        row < OUT_DIM,
        jnp.pad(whT, ((0, PR - OUT_DIM), (0, 0))),
        jnp.where(row == OUT_DIM, 1.0, 0.0),
    )
    vqk = jnp.concatenate(
        [jnp.exp2(tp), jnp.exp2(ALPHA * tp), jnp.exp2((ALPHA - 1.0) * sp)],
        axis=0,
    )
    params = jnp.where(
        (row >= NDW) & (row < NDW + 3),
        jnp.pad(vqk, ((NDW, PR - NDW - 3), (0, 0))),
        params,
    )
    params_ref[...] = params.astype(jnp.bfloat16)


def _gat_kernel(adj_ref, params_ref, out_ref):
    i = pl.program_id(0)
    v = params_ref[NDW : NDW + 1, :]
    q = params_ref[NDW + 1 : NDW + 2, :]
    k = jnp.reshape(
        params_ref[NDW + 2 : NDW + 3, pl.ds(i * BR, BR)], (BR, 1)
    )
    e = jnp.minimum(v, k * q) * adj_ref[...].astype(jnp.bfloat16)
    nd = jax.lax.dot_general(
        e, params_ref[:NDW, :], (((1,), (1,)), ((), ())),
        preferred_element_type=jnp.float32,
    )
    r = nd[:, :OUT_DIM] / nd[:, OUT_DIM : OUT_DIM + 1]
    out_ref[...] = jnp.where(r > 0.0, r, jnp.exp(jnp.minimum(r, 0.0)) - 1.0)


def kernel(input, adj_mat, weights, a_values):
    a_cols = a_values.reshape(2, OUT_DIM).T * (-1.4426950408889634)

    params = pl.pallas_call(
        _proj_kernel,
        out_shape=jax.ShapeDtypeStruct((PR, N), jnp.bfloat16),
    )(input, weights, a_cols)

    out = pl.pallas_call(
        _gat_kernel,
        grid=(N // BR,),
        in_specs=[
            pl.BlockSpec((BR, N), lambda i: (i, 0)),
            pl.BlockSpec((PR, N), lambda i: (0, 0)),
        ],
        out_specs=pl.BlockSpec((BR, OUT_DIM), lambda i: (i, 0)),
        out_shape=jax.ShapeDtypeStruct((N, OUT_DIM), jnp.float32),
        compiler_params=pltpu.CompilerParams(
            dimension_semantics=("arbitrary",)
        ),
    )(adj_mat, params)
    return out
